# baseline (speedup 1.0000x reference)
import functools

import jax
import jax.numpy as jnp
import numpy as np
from jax import lax
from jax.experimental import pallas as pl
from jax.experimental.pallas import tpu as pltpu
from jax.experimental.pallas import tpu_sc as plsc

EMBED = 768
DPT = EMBED // 4
MAX_SEQ = 24
BASE_GSD = 10
B, H, W, T, BS = 4, 16, 16, 12, 4
NROWS = B * H * W
CFLAT = BS * EMBED


def _sincos_1d(dim, pos):
    omega = jnp.arange(dim // 2, dtype=jnp.float32) / (dim / 2.0)
    omega = 1.0 / (10000.0 ** omega)
    out = pos.reshape(-1).astype(jnp.float32)[:, None] * omega[None, :]
    return jnp.concatenate([jnp.sin(out), jnp.cos(out)], axis=1)


def _pos_table():
    return _sincos_1d(DPT, jnp.arange(MAX_SEQ, dtype=jnp.float32))[:T]


def _month_table():
    angles = np.arange(0, 13) / (12 / (2 * np.pi))
    sin_table = np.sin(np.stack([angles for _ in range(DPT // 2)], axis=-1))
    cos_table = np.cos(np.stack([angles for _ in range(DPT // 2)], axis=-1))
    month_table = np.concatenate([sin_table[:-1], cos_table[:-1]], axis=-1)
    return jnp.asarray(month_table, dtype=jnp.float32)


def _spatial_table(gsd_ratio):
    grid_h = jnp.arange(H, dtype=jnp.float32)
    grid_w = jnp.arange(W, dtype=jnp.float32)
    gw, gh = jnp.meshgrid(grid_w, grid_h, indexing="xy")
    emb_h = _sincos_1d(DPT // 2, gw * gsd_ratio)
    emb_w = _sincos_1d(DPT // 2, gh * gsd_ratio)
    emb = jnp.concatenate([emb_h, emb_w], axis=1)
    return emb.reshape(H, W, DPT)


_SC_WORKERS = B
_GATHER_D = 256


def _month_gather(month_tab_padded, timestamps):
    mesh = plsc.VectorSubcoreMesh(core_axis_name="c", subcore_axis_name="s",
                                  num_cores=1)

    @functools.partial(
        pl.kernel,
        mesh=mesh,
        out_type=jax.ShapeDtypeStruct((B, 16, _GATHER_D), jnp.float32),
        scratch_types=[
            pltpu.VMEM((16,), jnp.int32),
            pltpu.VMEM((16, _GATHER_D), jnp.float32),
            pltpu.SemaphoreType.DMA,
        ],
    )
    def k(tab_hbm, m2_hbm, out_hbm, idx_v, rows_v, sem):
        wid = lax.axis_index("s")

        @pl.when(wid < _SC_WORKERS)
        def _():
            pltpu.sync_copy(m2_hbm.at[wid], idx_v)
            pltpu.async_copy(tab_hbm.at[idx_v], rows_v, sem).wait()
            pltpu.sync_copy(rows_v, out_hbm.at[wid])

    return k(month_tab_padded, timestamps)


_HG = 4


def _add_kernel(tok_ref, m_ref, ch_ref, pos_ref, q_ref, out_ref):
    hb = H // _HG
    j = pl.program_id(0)
    b = j // hb
    tok = tok_ref[0]
    m12 = m_ref[b, 0:T, 0:DPT]
    p = jnp.concatenate([
        jnp.broadcast_to(ch_ref[...][None], (T, BS, DPT)),
        jnp.broadcast_to(pos_ref[...][:, None, :], (T, BS, DPT)),
        jnp.broadcast_to(m12[:, None, :], (T, BS, DPT)),
        jnp.zeros((T, BS, DPT), jnp.float32),
    ], axis=-1)
    q = q_ref[pl.ds((j % hb) * _HG, _HG)]
    out_ref[0] = tok + p[None, None] + q


def _broadcast_add(tokens, month_out, channel_embed, pos, q_tab):
    hb = H // _HG
    return pl.pallas_call(
        _add_kernel,
        grid=(B * hb,),
        in_specs=[
            pl.BlockSpec((1, _HG, W, T, BS, EMBED),
                         lambda j: (j // hb, j % hb, 0, 0, 0, 0)),
            pl.BlockSpec((B, 16, _GATHER_D), lambda j: (0, 0, 0)),
            pl.BlockSpec((BS, DPT), lambda j: (0, 0)),
            pl.BlockSpec((T, DPT), lambda j: (0, 0)),
            pl.BlockSpec((H, W, 1, 1, EMBED), lambda j: (0, 0, 0, 0, 0)),
        ],
        out_specs=pl.BlockSpec((1, _HG, W, T, BS, EMBED),
                               lambda j: (j // hb, j % hb, 0, 0, 0, 0)),
        out_shape=jax.ShapeDtypeStruct((B, H, W, T, BS, EMBED), jnp.float32),
        compiler_params=pltpu.CompilerParams(
            dimension_semantics=("parallel",),
        ),
    )(tokens, month_out, channel_embed, pos, q_tab)


def kernel(tokens, timestamps, patch_size, input_res, channel_embed):
    tab_padded = jnp.pad(_month_table(), ((0, 0), (0, _GATHER_D - DPT)))
    months2 = jnp.pad(timestamps[:, :, 1].astype(jnp.int32),
                      ((0, 0), (0, 16 - T)))
    month_out = _month_gather(tab_padded, months2)

    gsd_ratio = (jnp.asarray(input_res, jnp.float32)
                 * jnp.asarray(patch_size, jnp.float32) / BASE_GSD)
    sp = _spatial_table(gsd_ratio)
    pos = _pos_table()

    q_tab = jnp.concatenate(
        [jnp.zeros((H, W, 3 * DPT), jnp.float32), sp], axis=-1)
    q_tab = q_tab[:, :, None, None, :]

    return _broadcast_add(tokens, month_out, channel_embed, pos, q_tab)


# speedup vs baseline: 1.0077x; 1.0077x over previous
import functools

import jax
import jax.numpy as jnp
import numpy as np
from jax import lax
from jax.experimental import pallas as pl
from jax.experimental.pallas import tpu as pltpu
from jax.experimental.pallas import tpu_sc as plsc

EMBED = 768
DPT = EMBED // 4
MAX_SEQ = 24
BASE_GSD = 10
B, H, W, T, BS = 4, 16, 16, 12, 4
NROWS = B * H * W
CFLAT = BS * EMBED


def _sincos_1d(dim, pos):
    omega = jnp.arange(dim // 2, dtype=jnp.float32) / (dim / 2.0)
    omega = 1.0 / (10000.0 ** omega)
    out = pos.reshape(-1).astype(jnp.float32)[:, None] * omega[None, :]
    return jnp.concatenate([jnp.sin(out), jnp.cos(out)], axis=1)


def _pos_table():
    return _sincos_1d(DPT, jnp.arange(MAX_SEQ, dtype=jnp.float32))[:T]


def _month_table():
    angles = np.arange(0, 13) / (12 / (2 * np.pi))
    sin_table = np.sin(np.stack([angles for _ in range(DPT // 2)], axis=-1))
    cos_table = np.cos(np.stack([angles for _ in range(DPT // 2)], axis=-1))
    month_table = np.concatenate([sin_table[:-1], cos_table[:-1]], axis=-1)
    return jnp.asarray(month_table, dtype=jnp.float32)


def _spatial_table(gsd_ratio):
    grid_h = jnp.arange(H, dtype=jnp.float32)
    grid_w = jnp.arange(W, dtype=jnp.float32)
    gw, gh = jnp.meshgrid(grid_w, grid_h, indexing="xy")
    emb_h = _sincos_1d(DPT // 2, gw * gsd_ratio)
    emb_w = _sincos_1d(DPT // 2, gh * gsd_ratio)
    emb = jnp.concatenate([emb_h, emb_w], axis=1)
    return emb.reshape(H, W, DPT)


_SC_WORKERS = B
_GATHER_D = 256


def _month_gather(month_tab_padded, timestamps):
    mesh = plsc.VectorSubcoreMesh(core_axis_name="c", subcore_axis_name="s",
                                  num_cores=1)

    @functools.partial(
        pl.kernel,
        mesh=mesh,
        out_type=jax.ShapeDtypeStruct((B, 16, _GATHER_D), jnp.float32),
        scratch_types=[
            pltpu.VMEM((16,), jnp.int32),
            pltpu.VMEM((16, _GATHER_D), jnp.float32),
            pltpu.SemaphoreType.DMA,
        ],
    )
    def k(tab_hbm, m2_hbm, out_hbm, idx_v, rows_v, sem):
        wid = lax.axis_index("s")

        @pl.when(wid < _SC_WORKERS)
        def _():
            pltpu.sync_copy(m2_hbm.at[wid], idx_v)
            pltpu.async_copy(tab_hbm.at[idx_v], rows_v, sem).wait()
            pltpu.sync_copy(rows_v, out_hbm.at[wid])

    return k(month_tab_padded, timestamps)


_HG = 4


def _add_kernel(tok_ref, m_ref, ch_ref, pos_ref, q_ref, out_ref):
    hb = H // _HG
    j = pl.program_id(0)
    b = j // hb
    tok = tok_ref[0]
    m12 = m_ref[b, 0:T, 0:DPT]
    p = jnp.concatenate([
        jnp.broadcast_to(ch_ref[...][None], (T, BS, DPT)),
        jnp.broadcast_to(pos_ref[...][:, None, :], (T, BS, DPT)),
        jnp.broadcast_to(m12[:, None, :], (T, BS, DPT)),
        jnp.zeros((T, BS, DPT), jnp.float32),
    ], axis=-1)
    q = q_ref[pl.ds((j % hb) * _HG, _HG)]
    out_ref[0] = tok + p[None, None] + q


def _broadcast_add(tokens, month_out, channel_embed, pos, q_tab):
    hb = H // _HG
    return pl.pallas_call(
        _add_kernel,
        grid=(B * hb,),
        in_specs=[
            pl.BlockSpec((1, _HG, W, T, BS, EMBED),
                         lambda j: (j // hb, j % hb, 0, 0, 0, 0)),
            pl.BlockSpec((B, 16, _GATHER_D), lambda j: (0, 0, 0)),
            pl.BlockSpec((BS, DPT), lambda j: (0, 0)),
            pl.BlockSpec((T, DPT), lambda j: (0, 0)),
            pl.BlockSpec((H, W, 1, 1, EMBED), lambda j: (0, 0, 0, 0, 0)),
        ],
        out_specs=pl.BlockSpec((1, _HG, W, T, BS, EMBED),
                               lambda j: (j // hb, j % hb, 0, 0, 0, 0)),
        out_shape=jax.ShapeDtypeStruct((B, H, W, T, BS, EMBED), jnp.float32),
        compiler_params=pltpu.CompilerParams(
            dimension_semantics=("arbitrary",),
        ),
    )(tokens, month_out, channel_embed, pos, q_tab)


def kernel(tokens, timestamps, patch_size, input_res, channel_embed):
    tab_padded = jnp.pad(_month_table(), ((0, 0), (0, _GATHER_D - DPT)))
    months2 = jnp.pad(timestamps[:, :, 1].astype(jnp.int32),
                      ((0, 0), (0, 16 - T)))
    month_out = _month_gather(tab_padded, months2)

    gsd_ratio = (jnp.asarray(input_res, jnp.float32)
                 * jnp.asarray(patch_size, jnp.float32) / BASE_GSD)
    sp = _spatial_table(gsd_ratio)
    pos = _pos_table()

    q_tab = jnp.concatenate(
        [jnp.zeros((H, W, 3 * DPT), jnp.float32), sp], axis=-1)
    q_tab = q_tab[:, :, None, None, :]

    return _broadcast_add(tokens, month_out, channel_embed, pos, q_tab)
